# baseline (speedup 1.0000x reference)
import functools

import jax
import jax.numpy as jnp
from jax import lax
from jax.experimental import pallas as pl
from jax.experimental.pallas import tpu as pltpu
from jax.experimental.pallas import tpu_sc as plsc

B = 8
MAX_LEN = 2048
D_MODEL = 1024
CHUNK = 64

_info = plsc.get_sparse_core_info()
NC, NS = _info.num_cores, _info.num_subcores
NW = NC * NS
RPW = B * MAX_LEN // NW
SPANS_PER_BATCH = MAX_LEN // RPW


def _pad_body(flat_hbm, cu_hbm, zeros_hbm, out_hbm, cu_v):
    wid = lax.axis_index("s") * NC + lax.axis_index("c")
    b = wid // SPANS_PER_BATCH
    j0 = (wid % SPANS_PER_BATCH) * RPW
    dst0 = wid * RPW

    pltpu.sync_copy(cu_hbm, cu_v)
    cuvec = cu_v[...]
    cu_b = jnp.int32(0)
    cu_b1 = jnp.int32(0)
    for i in range(B + 1):
        ci = cuvec[i]
        cu_b = jnp.where(b == i, ci, cu_b)
        cu_b1 = jnp.where(b + 1 == i, ci, cu_b1)
    vcnt = jnp.clip(cu_b1 - cu_b - j0, 0, RPW)
    src0 = cu_b + j0

    for c in range(RPW // CHUNK):
        rem = vcnt - c * CHUNK
        dst = dst0 + c * CHUNK

        @pl.when(rem >= CHUNK)
        def _copy_full():
            pltpu.sync_copy(
                flat_hbm.at[pl.ds((src0 + c * CHUNK) * D_MODEL, CHUNK * D_MODEL)],
                out_hbm.at[pl.ds(dst * D_MODEL, CHUNK * D_MODEL)],
            )

        @pl.when(rem < CHUNK)
        def _zero_fill():
            pltpu.sync_copy(
                zeros_hbm, out_hbm.at[pl.ds(dst * D_MODEL, CHUNK * D_MODEL)]
            )

        @pl.when(jnp.logical_and(rem > 0, rem < CHUNK))
        def _overlay_partial():
            off = jnp.int32(0)
            src = src0 + c * CHUNK
            for sz in (32, 16, 8, 4, 2, 1):
                bit = (rem & sz) != 0

                @pl.when(bit)
                def _copy_piece(off=off, sz=sz):
                    pltpu.sync_copy(
                        flat_hbm.at[pl.ds((src + off) * D_MODEL, sz * D_MODEL)],
                        out_hbm.at[pl.ds((dst + off) * D_MODEL, sz * D_MODEL)],
                    )

                off = off + jnp.where(bit, sz, 0).astype(jnp.int32)


@functools.partial(jax.jit, static_argnums=())
def _pad_call(flat, cu16, zeros):
    mesh = plsc.VectorSubcoreMesh(core_axis_name="c", subcore_axis_name="s")
    fn = functools.partial(
        pl.kernel,
        mesh=mesh,
        out_type=jax.ShapeDtypeStruct((B * MAX_LEN * D_MODEL,), flat.dtype),
        scratch_types=[pltpu.VMEM((16,), jnp.int32)],
    )(_pad_body)
    return fn(flat, cu16, zeros)


def kernel(flat, cu_seqlens):
    cu16 = jnp.zeros((16,), jnp.int32).at[: cu_seqlens.shape[0]].set(cu_seqlens)
    zeros = jnp.zeros((CHUNK * D_MODEL,), flat.dtype)
    out = _pad_call(flat.reshape(-1), cu16, zeros)
    return out.reshape(B, MAX_LEN, D_MODEL)


# speedup vs baseline: 12.8908x; 12.8908x over previous
import functools

import jax
import jax.numpy as jnp
from jax import lax
from jax.experimental import pallas as pl
from jax.experimental.pallas import tpu as pltpu
from jax.experimental.pallas import tpu_sc as plsc

B = 8
MAX_LEN = 2048
D_MODEL = 1024
CHUNK = 32

_info = plsc.get_sparse_core_info()
NC, NS = _info.num_cores, _info.num_subcores
NW = NC * NS
RPW = B * MAX_LEN // NW
NCHUNK = RPW // CHUNK
SPANS_PER_BATCH = MAX_LEN // RPW
CD = CHUNK * D_MODEL


def _pad_body(flat_hbm, cu_hbm, zeros_hbm, out_hbm, cu_v, buf0, buf1, zbuf,
              ssem0, ssem1):
    wid = lax.axis_index("s") * NC + lax.axis_index("c")
    b = wid // SPANS_PER_BATCH
    j0 = (wid % SPANS_PER_BATCH) * RPW
    dst0 = wid * RPW

    pltpu.sync_copy(cu_hbm, cu_v)
    pltpu.sync_copy(zeros_hbm, zbuf)
    cuvec = cu_v[...]
    cu_b = jnp.int32(0)
    cu_b1 = jnp.int32(0)
    for i in range(B + 1):
        ci = cuvec[i]
        cu_b = jnp.where(b == i, ci, cu_b)
        cu_b1 = jnp.where(b + 1 == i, ci, cu_b1)
    vcnt = jnp.clip(cu_b1 - cu_b - j0, 0, RPW)
    src0 = cu_b + j0

    bufs = (buf0, buf1)
    ssems = (ssem0, ssem1)

    for c in range(NCHUNK):
        p = c % 2
        buf, ssem = bufs[p], ssems[p]
        rem = vcnt - c * CHUNK
        dst = dst0 + c * CHUNK

        if c >= 2:
            pltpu.make_async_copy(zeros_hbm, buf, ssem).wait()

        @pl.when(rem >= CHUNK)
        def _gather_full():
            pltpu.sync_copy(
                flat_hbm.at[pl.ds((src0 + c * CHUNK) * D_MODEL, CD)], buf
            )

        @pl.when(jnp.logical_and(rem > 0, rem < CHUNK))
        def _assemble_partial():
            pltpu.sync_copy(zeros_hbm, buf)
            off = jnp.int32(0)
            src = src0 + c * CHUNK
            for sz in (16, 8, 4, 2, 1):
                bit = (rem & sz) != 0

                @pl.when(bit)
                def _gather_piece(off=off, sz=sz):
                    pltpu.sync_copy(
                        flat_hbm.at[pl.ds((src + off) * D_MODEL, sz * D_MODEL)],
                        buf.at[pl.ds(off * D_MODEL, sz * D_MODEL)],
                    )

                off = off + jnp.where(bit, sz, 0).astype(jnp.int32)

        @pl.when(rem > 0)
        def _scatter_data():
            pltpu.async_copy(buf, out_hbm.at[pl.ds(dst * D_MODEL, CD)], ssem)

        @pl.when(rem <= 0)
        def _scatter_zero():
            pltpu.async_copy(zbuf, out_hbm.at[pl.ds(dst * D_MODEL, CD)], ssem)

    pltpu.make_async_copy(zeros_hbm, buf0, ssem0).wait()
    pltpu.make_async_copy(zeros_hbm, buf1, ssem1).wait()


@jax.jit
def _pad_call(flat, cu16, zeros):
    mesh = plsc.VectorSubcoreMesh(core_axis_name="c", subcore_axis_name="s")
    fn = functools.partial(
        pl.kernel,
        mesh=mesh,
        out_type=jax.ShapeDtypeStruct((B * MAX_LEN * D_MODEL,), flat.dtype),
        scratch_types=[
            pltpu.VMEM((16,), jnp.int32),
            pltpu.VMEM((CD,), jnp.float32),
            pltpu.VMEM((CD,), jnp.float32),
            pltpu.VMEM((CD,), jnp.float32),
            pltpu.SemaphoreType.DMA,
            pltpu.SemaphoreType.DMA,
        ],
    )(_pad_body)
    return fn(flat, cu16, zeros)


def kernel(flat, cu_seqlens):
    cu16 = jnp.zeros((16,), jnp.int32).at[: cu_seqlens.shape[0]].set(cu_seqlens)
    zeros = jnp.zeros((CD,), flat.dtype)
    out = _pad_call(flat.reshape(-1), cu16, zeros)
    return out.reshape(B, MAX_LEN, D_MODEL)


# speedup vs baseline: 13.5904x; 1.0543x over previous
import functools

import jax
import jax.numpy as jnp
from jax import lax
from jax.experimental import pallas as pl
from jax.experimental.pallas import tpu as pltpu
from jax.experimental.pallas import tpu_sc as plsc

B = 8
MAX_LEN = 2048
D_MODEL = 1024
CHUNK = 32

_info = plsc.get_sparse_core_info()
NC, NS = _info.num_cores, _info.num_subcores
NW = NC * NS
TOT_CHUNKS = B * MAX_LEN // CHUNK
CPW = TOT_CHUNKS // NW
CHUNKS_PER_BATCH = MAX_LEN // CHUNK
CD = CHUNK * D_MODEL
HCD = CD // 2
NBUF = 3
PF = 2


def _pad_body(flat_hbm, cu_hbm, zeros_hbm, out_hbm, cu_v, ring0, ring1, ring2,
              zbuf, gsem0, gsem1, gsem2, ssem0, ssem1, ssem2):
    wid = lax.axis_index("s") * NC + lax.axis_index("c")

    pltpu.sync_copy(cu_hbm, cu_v)
    pltpu.sync_copy(zeros_hbm, zbuf)
    cuvec = cu_v[...]
    cus = [cuvec[i] for i in range(B + 1)]

    rings = (ring0, ring1, ring2)
    gsems = (gsem0, gsem1, gsem2)
    ssems = (ssem0, ssem1, ssem2)

    def params(t):
        k = wid + t * NW
        b = k // CHUNKS_PER_BATCH
        j = (k % CHUNKS_PER_BATCH) * CHUNK
        cu_b = jnp.int32(0)
        cu_b1 = jnp.int32(0)
        for i in range(B + 1):
            cu_b = jnp.where(b == i, cus[i], cu_b)
            cu_b1 = jnp.where(b + 1 == i, cus[i], cu_b1)
        rem = cu_b1 - cu_b - j
        src = cu_b + j
        return k, src, rem

    def maybe_gather(t):
        if t >= CPW:
            return
        p = t % NBUF
        if t >= NBUF:
            pltpu.make_async_copy(
                flat_hbm.at[pl.ds(0, CD)], rings[p], ssems[p]
            ).wait()
        _, src, rem = params(t)

        @pl.when(rem >= CHUNK)
        def _gather_full():
            pltpu.async_copy(
                flat_hbm.at[pl.ds(src * D_MODEL, CD)], rings[p], gsems[p]
            )

        @pl.when(jnp.logical_and(rem > 0, rem < CHUNK))
        def _assemble_partial():
            pltpu.sync_copy(zeros_hbm, rings[p].at[pl.ds(0, HCD)])
            pltpu.sync_copy(zeros_hbm, rings[p].at[pl.ds(HCD, HCD)])
            off = jnp.int32(0)
            for sz in (16, 8, 4, 2, 1):
                bit = (rem & sz) != 0

                @pl.when(bit)
                def _gather_piece(off=off, sz=sz):
                    pltpu.sync_copy(
                        flat_hbm.at[pl.ds((src + off) * D_MODEL, sz * D_MODEL)],
                        rings[p].at[pl.ds(off * D_MODEL, sz * D_MODEL)],
                    )

                off = off + jnp.where(bit, sz, 0).astype(jnp.int32)
            pltpu.async_copy(zeros_hbm, zbuf, gsems[p])
            pltpu.async_copy(zeros_hbm, zbuf, gsems[p])

    def consume(t):
        p = t % NBUF
        k, src, rem = params(t)
        dst = k * CHUNK * D_MODEL

        @pl.when(rem > 0)
        def _wait_and_scatter_data():
            pltpu.make_async_copy(
                flat_hbm.at[pl.ds(0, CD)], rings[p], gsems[p]
            ).wait()
            pltpu.async_copy(rings[p], out_hbm.at[pl.ds(dst, CD)], ssems[p])

        @pl.when(rem <= 0)
        def _scatter_zero():
            pltpu.async_copy(zbuf, out_hbm.at[pl.ds(dst, HCD)], ssems[p])
            pltpu.async_copy(
                zbuf, out_hbm.at[pl.ds(dst + HCD, HCD)], ssems[p]
            )

    for g in range(PF):
        maybe_gather(g)
    for t in range(CPW):
        maybe_gather(t + PF)
        consume(t)
    for t in range(CPW - NBUF, CPW):
        p = t % NBUF
        pltpu.make_async_copy(
            flat_hbm.at[pl.ds(0, CD)], rings[p], ssems[p]
        ).wait()


@jax.jit
def _pad_call(flat, cu16, zeros):
    mesh = plsc.VectorSubcoreMesh(core_axis_name="c", subcore_axis_name="s")
    fn = functools.partial(
        pl.kernel,
        mesh=mesh,
        out_type=jax.ShapeDtypeStruct((B * MAX_LEN * D_MODEL,), flat.dtype),
        scratch_types=[
            pltpu.VMEM((16,), jnp.int32),
            pltpu.VMEM((CD,), jnp.float32),
            pltpu.VMEM((CD,), jnp.float32),
            pltpu.VMEM((CD,), jnp.float32),
            pltpu.VMEM((HCD,), jnp.float32),
            pltpu.SemaphoreType.DMA,
            pltpu.SemaphoreType.DMA,
            pltpu.SemaphoreType.DMA,
            pltpu.SemaphoreType.DMA,
            pltpu.SemaphoreType.DMA,
            pltpu.SemaphoreType.DMA,
        ],
    )(_pad_body)
    return fn(flat, cu16, zeros)


def kernel(flat, cu_seqlens):
    cu16 = jnp.zeros((16,), jnp.int32).at[: cu_seqlens.shape[0]].set(cu_seqlens)
    zeros = jnp.zeros((HCD,), flat.dtype)
    out = _pad_call(flat.reshape(-1), cu16, zeros)
    return out.reshape(B, MAX_LEN, D_MODEL)
